# baseline (speedup 1.0000x reference)
import functools

import jax
import jax.numpy as jnp
from jax.experimental import pallas as pl
from jax.experimental.pallas import tpu as pltpu

_IGNORE_INDEX = -100
_LANES = 1024
_TILE_ROWS = 512
_CORES = 2


def _round_up(x, m):
    return ((x + m - 1) // m) * m


def _vcl_kernel(logp_ref, tgt_ref, mu_ref, lv_ref, mu_o_ref, lv_o_ref,
                out_ref, acc_ref, nll_ref, *,
                kt, tile_rows, kl_scale, split_nll, valid_rows, needs_mask):
    c = pl.program_id(0)
    k = pl.program_id(1)
    row0 = (c * kt + k) * tile_rows

    do_nll = (k == 0) if split_nll else jnp.logical_and(k == 0, c == 0)

    @pl.when(k == 0)
    def _init():
        acc_ref[...] = jnp.zeros_like(acc_ref)
        nll_ref[0] = 0.0
        nll_ref[1] = 0.0

    @pl.when(do_nll)
    def _nll_partial():
        logp = logp_ref[...].astype(jnp.float32)
        tgt = tgt_ref[...]
        n, ncls = logp.shape
        cls = jax.lax.broadcasted_iota(jnp.int32, (n, ncls), 1)
        valid = tgt != _IGNORE_INDEX
        hit = jnp.logical_and(cls == tgt, valid)
        nll_ref[0] = jnp.sum(jnp.where(hit, logp, 0.0))
        nll_ref[1] = jnp.sum(valid.astype(jnp.float32))

    mu = mu_ref[...].astype(jnp.float32)
    lv = lv_ref[...].astype(jnp.float32)
    mu_o = mu_o_ref[...].astype(jnp.float32)
    lv_o = lv_o_ref[...].astype(jnp.float32)
    t = lv_o - lv + jnp.exp(lv - lv_o) + jnp.square(mu - mu_o) * jnp.exp(-lv_o) - 1.0
    if needs_mask:
        ridx = jax.lax.broadcasted_iota(jnp.int32, t.shape, 0)
        t = jnp.where(ridx < (valid_rows - row0), t, 0.0)
    r, w = t.shape
    acc_ref[...] += jnp.sum(t.reshape(r // 8, 8, w), axis=0)

    @pl.when(k == kt - 1)
    def _finalize():
        kl_sum = (0.5 * kl_scale) * jnp.sum(acc_ref[...])
        lane = jax.lax.broadcasted_iota(jnp.int32, (8, 128), 1)
        sub = jax.lax.broadcasted_iota(jnp.int32, (8, 128), 0)
        vals = jnp.where(lane == 0, kl_sum,
               jnp.where(lane == 1, nll_ref[0],
               jnp.where(lane == 2, nll_ref[1], 0.0)))
        out_ref[0] = jnp.where(sub == 0, vals, 0.0)


def kernel(output, target, mu_new, lv_new, mu_old, lv_old):
    n, ncls = output.shape
    tgt2d = target.reshape(n, 1).astype(jnp.int32)
    kl_scale = 1.0 / float(n)

    nelem = mu_new.size
    lanes = _LANES
    rows = _round_up(nelem, lanes) // lanes

    def to_rows(a):
        flat = jnp.ravel(a)
        if nelem % lanes:
            flat = jnp.pad(flat, (0, rows * lanes - nelem))
        return flat.reshape(rows, lanes)

    slabs = [to_rows(a) for a in (mu_new, lv_new, mu_old, lv_old)]

    def plan(num_cores):
        rpc = pl.cdiv(rows, num_cores)
        tr = min(_TILE_ROWS, _round_up(rpc, 8))
        return tr, pl.cdiv(rpc, tr)

    nc = _CORES
    tile_rows, kt = plan(nc)
    if nc > 1 and kt * tile_rows >= rows:
        nc = 1
        tile_rows, kt = plan(nc)

    needs_mask = (nc * kt * tile_rows != rows)
    max_block = pl.cdiv(rows, tile_rows) - 1

    def slab_map(cc, kk):
        return (jnp.minimum(cc * kt + kk, max_block), 0)

    slab_spec = pl.BlockSpec((tile_rows, lanes), slab_map)

    split_nll = nc > 1 and n % nc == 0 and (n // nc) % 8 == 0
    if split_nll:
        n_blk = n // nc
        logp_spec = pl.BlockSpec((n_blk, ncls), lambda cc, kk: (cc, 0))
        tgt_spec = pl.BlockSpec((n_blk, 1), lambda cc, kk: (cc, 0))
    else:
        logp_spec = pl.BlockSpec((n, ncls), lambda cc, kk: (0, 0))
        tgt_spec = pl.BlockSpec((n, 1), lambda cc, kk: (0, 0))

    _kernel_fn = functools.partial(
        _vcl_kernel, kt=kt, tile_rows=tile_rows, kl_scale=kl_scale,
        split_nll=split_nll, valid_rows=rows, needs_mask=needs_mask)

    bytes_accessed = int(sum(s.size * s.dtype.itemsize for s in slabs)
                         + output.size * output.dtype.itemsize
                         + tgt2d.size * tgt2d.dtype.itemsize
                         + nc * 8 * 128 * 4)
    cost = pl.CostEstimate(flops=int(9 * nelem + 4 * n * ncls),
                           transcendentals=int(2 * nelem),
                           bytes_accessed=bytes_accessed)

    out = pl.pallas_call(
        _kernel_fn,
        out_shape=jax.ShapeDtypeStruct((nc, 8, 128), jnp.float32),
        grid=(nc, kt),
        in_specs=[logp_spec, tgt_spec,
                  slab_spec, slab_spec, slab_spec, slab_spec],
        out_specs=pl.BlockSpec((1, 8, 128), lambda cc, kk: (cc, 0, 0)),
        scratch_shapes=[pltpu.VMEM((8, lanes), jnp.float32),
                        pltpu.SMEM((2,), jnp.float32)],
        compiler_params=pltpu.CompilerParams(
            dimension_semantics=("parallel", "arbitrary")),
        cost_estimate=cost,
    )(output, tgt2d, *slabs)

    kl = jnp.sum(out[:, 0, 0])
    picked = jnp.sum(out[:, 0, 1])
    nvalid = jnp.maximum(jnp.sum(out[:, 0, 2]), 1.0)
    return kl - picked / nvalid


# speedup vs baseline: 3.5250x; 3.5250x over previous
import functools

import jax
import jax.numpy as jnp
from jax.experimental import pallas as pl
from jax.experimental.pallas import tpu as pltpu

_IGNORE_INDEX = -100
_LANES = 128
_TILE_ROWS = 4096
_CORES = 2


def _round_up(x, m):
    return ((x + m - 1) // m) * m


def _vcl_kernel(logp_ref, tgt_ref, mu_ref, lv_ref, mu_o_ref, lv_o_ref,
                out_ref, acc_ref, nll_ref, *,
                kt, tile_rows, kl_scale, split_nll, valid_rows, needs_mask):
    c = pl.program_id(0)
    k = pl.program_id(1)
    row0 = (c * kt + k) * tile_rows

    do_nll = (k == 0) if split_nll else jnp.logical_and(k == 0, c == 0)

    @pl.when(k == 0)
    def _init():
        acc_ref[...] = jnp.zeros_like(acc_ref)
        nll_ref[0] = 0.0
        nll_ref[1] = 0.0

    @pl.when(do_nll)
    def _nll_partial():
        logp = logp_ref[...].astype(jnp.float32)
        tgt = tgt_ref[...]
        n, ncls = logp.shape
        cls = jax.lax.broadcasted_iota(jnp.int32, (n, ncls), 1)
        valid = tgt != _IGNORE_INDEX
        hit = jnp.logical_and(cls == tgt, valid)
        nll_ref[0] = jnp.sum(jnp.where(hit, logp, 0.0))
        nll_ref[1] = jnp.sum(valid.astype(jnp.float32))

    mu = mu_ref[...].astype(jnp.float32)
    lv = lv_ref[...].astype(jnp.float32)
    mu_o = mu_o_ref[...].astype(jnp.float32)
    lv_o = lv_o_ref[...].astype(jnp.float32)
    t = lv_o - lv + jnp.exp(lv - lv_o) + jnp.square(mu - mu_o) * jnp.exp(-lv_o) - 1.0
    if needs_mask:
        ridx = jax.lax.broadcasted_iota(jnp.int32, t.shape, 0)
        t = jnp.where(ridx < (valid_rows - row0), t, 0.0)
    r, w = t.shape
    acc_ref[...] += jnp.sum(t.reshape(r // 8, 8, w), axis=0)

    @pl.when(k == kt - 1)
    def _finalize():
        kl_sum = (0.5 * kl_scale) * jnp.sum(acc_ref[...])
        lane = jax.lax.broadcasted_iota(jnp.int32, (8, 128), 1)
        sub = jax.lax.broadcasted_iota(jnp.int32, (8, 128), 0)
        vals = jnp.where(lane == 0, kl_sum,
               jnp.where(lane == 1, nll_ref[0],
               jnp.where(lane == 2, nll_ref[1], 0.0)))
        out_ref[0] = jnp.where(sub == 0, vals, 0.0)


def kernel(output, target, mu_new, lv_new, mu_old, lv_old):
    n, ncls = output.shape
    tgt2d = target.reshape(n, 1).astype(jnp.int32)
    kl_scale = 1.0 / float(n)

    nelem = mu_new.size
    lanes = _LANES
    rows = _round_up(nelem, lanes) // lanes

    def to_rows(a):
        flat = jnp.ravel(a)
        if nelem % lanes:
            flat = jnp.pad(flat, (0, rows * lanes - nelem))
        return flat.reshape(rows, lanes)

    slabs = [to_rows(a) for a in (mu_new, lv_new, mu_old, lv_old)]

    def plan(num_cores):
        rpc = pl.cdiv(rows, num_cores)
        tr = min(_TILE_ROWS, _round_up(rpc, 8))
        return tr, pl.cdiv(rpc, tr)

    nc = _CORES
    tile_rows, kt = plan(nc)
    if nc > 1 and kt * tile_rows >= rows:
        nc = 1
        tile_rows, kt = plan(nc)

    needs_mask = (nc * kt * tile_rows != rows)
    max_block = pl.cdiv(rows, tile_rows) - 1

    def slab_map(cc, kk):
        return (jnp.minimum(cc * kt + kk, max_block), 0)

    slab_spec = pl.BlockSpec((tile_rows, lanes), slab_map)

    split_nll = nc > 1 and n % nc == 0 and (n // nc) % 8 == 0
    if split_nll:
        n_blk = n // nc
        logp_spec = pl.BlockSpec((n_blk, ncls), lambda cc, kk: (cc, 0))
        tgt_spec = pl.BlockSpec((n_blk, 1), lambda cc, kk: (cc, 0))
    else:
        logp_spec = pl.BlockSpec((n, ncls), lambda cc, kk: (0, 0))
        tgt_spec = pl.BlockSpec((n, 1), lambda cc, kk: (0, 0))

    _kernel_fn = functools.partial(
        _vcl_kernel, kt=kt, tile_rows=tile_rows, kl_scale=kl_scale,
        split_nll=split_nll, valid_rows=rows, needs_mask=needs_mask)

    bytes_accessed = int(sum(s.size * s.dtype.itemsize for s in slabs)
                         + output.size * output.dtype.itemsize
                         + tgt2d.size * tgt2d.dtype.itemsize
                         + nc * 8 * 128 * 4)
    cost = pl.CostEstimate(flops=int(9 * nelem + 4 * n * ncls),
                           transcendentals=int(2 * nelem),
                           bytes_accessed=bytes_accessed)

    out = pl.pallas_call(
        _kernel_fn,
        out_shape=jax.ShapeDtypeStruct((nc, 8, 128), jnp.float32),
        grid=(nc, kt),
        in_specs=[logp_spec, tgt_spec,
                  slab_spec, slab_spec, slab_spec, slab_spec],
        out_specs=pl.BlockSpec((1, 8, 128), lambda cc, kk: (cc, 0, 0)),
        scratch_shapes=[pltpu.VMEM((8, lanes), jnp.float32),
                        pltpu.SMEM((2,), jnp.float32)],
        compiler_params=pltpu.CompilerParams(
            dimension_semantics=("parallel", "arbitrary")),
        cost_estimate=cost,
    )(output, tgt2d, *slabs)

    kl = jnp.sum(out[:, 0, 0])
    picked = jnp.sum(out[:, 0, 1])
    nvalid = jnp.maximum(jnp.sum(out[:, 0, 2]), 1.0)
    return kl - picked / nvalid


# speedup vs baseline: 3.7562x; 1.0656x over previous
import functools

import jax
import jax.numpy as jnp
from jax.experimental import pallas as pl
from jax.experimental.pallas import tpu as pltpu

_IGNORE_INDEX = -100
_LANES = 128
_TILE_ROWS = 8192
_CORES = 2


def _round_up(x, m):
    return ((x + m - 1) // m) * m


def _vcl_kernel(logp_ref, tgt_ref, mu_ref, lv_ref, mu_o_ref, lv_o_ref,
                out_ref, acc_ref, nll_ref, *,
                kt, tile_rows, kl_scale, split_nll, valid_rows, needs_mask):
    c = pl.program_id(0)
    k = pl.program_id(1)
    row0 = (c * kt + k) * tile_rows

    do_nll = (k == 0) if split_nll else jnp.logical_and(k == 0, c == 0)

    @pl.when(k == 0)
    def _init():
        acc_ref[...] = jnp.zeros_like(acc_ref)
        nll_ref[0] = 0.0
        nll_ref[1] = 0.0

    @pl.when(do_nll)
    def _nll_partial():
        logp = logp_ref[...].astype(jnp.float32)
        tgt = tgt_ref[...]
        n, ncls = logp.shape
        cls = jax.lax.broadcasted_iota(jnp.int32, (n, ncls), 1)
        valid = tgt != _IGNORE_INDEX
        hit = jnp.logical_and(cls == tgt, valid)
        nll_ref[0] = jnp.sum(jnp.where(hit, logp, 0.0))
        nll_ref[1] = jnp.sum(valid.astype(jnp.float32))

    mu = mu_ref[...].astype(jnp.float32)
    lv = lv_ref[...].astype(jnp.float32)
    mu_o = mu_o_ref[...].astype(jnp.float32)
    lv_o = lv_o_ref[...].astype(jnp.float32)
    t = lv_o - lv + jnp.exp(lv - lv_o) + jnp.square(mu - mu_o) * jnp.exp(-lv_o) - 1.0
    if needs_mask:
        ridx = jax.lax.broadcasted_iota(jnp.int32, t.shape, 0)
        t = jnp.where(ridx < (valid_rows - row0), t, 0.0)
    r, w = t.shape
    acc_ref[...] += jnp.sum(t.reshape(r // 8, 8, w), axis=0)

    @pl.when(k == kt - 1)
    def _finalize():
        kl_sum = (0.5 * kl_scale) * jnp.sum(acc_ref[...])
        lane = jax.lax.broadcasted_iota(jnp.int32, (8, 128), 1)
        sub = jax.lax.broadcasted_iota(jnp.int32, (8, 128), 0)
        vals = jnp.where(lane == 0, kl_sum,
               jnp.where(lane == 1, nll_ref[0],
               jnp.where(lane == 2, nll_ref[1], 0.0)))
        out_ref[0] = jnp.where(sub == 0, vals, 0.0)


def kernel(output, target, mu_new, lv_new, mu_old, lv_old):
    n, ncls = output.shape
    tgt2d = target.reshape(n, 1).astype(jnp.int32)
    kl_scale = 1.0 / float(n)

    nelem = mu_new.size
    lanes = _LANES
    rows = _round_up(nelem, lanes) // lanes

    def to_rows(a):
        flat = jnp.ravel(a)
        if nelem % lanes:
            flat = jnp.pad(flat, (0, rows * lanes - nelem))
        return flat.reshape(rows, lanes)

    slabs = [to_rows(a) for a in (mu_new, lv_new, mu_old, lv_old)]

    def plan(num_cores):
        rpc = pl.cdiv(rows, num_cores)
        tr = min(_TILE_ROWS, _round_up(rpc, 8))
        return tr, pl.cdiv(rpc, tr)

    nc = _CORES
    tile_rows, kt = plan(nc)
    if nc > 1 and kt * tile_rows >= rows:
        nc = 1
        tile_rows, kt = plan(nc)

    needs_mask = (nc * kt * tile_rows != rows)
    max_block = pl.cdiv(rows, tile_rows) - 1

    def slab_map(cc, kk):
        return (jnp.minimum(cc * kt + kk, max_block), 0)

    slab_spec = pl.BlockSpec((tile_rows, lanes), slab_map)

    split_nll = nc > 1 and n % nc == 0 and (n // nc) % 8 == 0
    if split_nll:
        n_blk = n // nc
        logp_spec = pl.BlockSpec((n_blk, ncls), lambda cc, kk: (cc, 0))
        tgt_spec = pl.BlockSpec((n_blk, 1), lambda cc, kk: (cc, 0))
    else:
        logp_spec = pl.BlockSpec((n, ncls), lambda cc, kk: (0, 0))
        tgt_spec = pl.BlockSpec((n, 1), lambda cc, kk: (0, 0))

    _kernel_fn = functools.partial(
        _vcl_kernel, kt=kt, tile_rows=tile_rows, kl_scale=kl_scale,
        split_nll=split_nll, valid_rows=rows, needs_mask=needs_mask)

    bytes_accessed = int(sum(s.size * s.dtype.itemsize for s in slabs)
                         + output.size * output.dtype.itemsize
                         + tgt2d.size * tgt2d.dtype.itemsize
                         + nc * 8 * 128 * 4)
    cost = pl.CostEstimate(flops=int(9 * nelem + 4 * n * ncls),
                           transcendentals=int(2 * nelem),
                           bytes_accessed=bytes_accessed)

    out = pl.pallas_call(
        _kernel_fn,
        out_shape=jax.ShapeDtypeStruct((nc, 8, 128), jnp.float32),
        grid=(nc, kt),
        in_specs=[logp_spec, tgt_spec,
                  slab_spec, slab_spec, slab_spec, slab_spec],
        out_specs=pl.BlockSpec((1, 8, 128), lambda cc, kk: (cc, 0, 0)),
        scratch_shapes=[pltpu.VMEM((8, lanes), jnp.float32),
                        pltpu.SMEM((2,), jnp.float32)],
        compiler_params=pltpu.CompilerParams(
            dimension_semantics=("parallel", "arbitrary")),
        cost_estimate=cost,
    )(output, tgt2d, *slabs)

    kl = jnp.sum(out[:, 0, 0])
    picked = jnp.sum(out[:, 0, 1])
    nvalid = jnp.maximum(jnp.sum(out[:, 0, 2]), 1.0)
    return kl - picked / nvalid
